# Initial kernel scaffold; baseline (speedup 1.0000x reference)
import functools

import jax
import jax.numpy as jnp
from jax import lax
from jax.experimental import pallas as pl
from jax.experimental.pallas import tpu as pltpu
from jax.experimental.pallas import tpu_sc as plsc

_B, _A, _C, _G = 8, 49104, 80, 20
_APAD = 49152
_NTILES = 32
_CHUNK = _APAD // _NTILES
_NVEC = _CHUNK // 16
_NGT = _B * _G
_ABLK = 1488
_NBLK = _A // _ABLK
_LN2 = 0.6931471805599453
_SQRT2 = 1.4142135623730951


def _sc_log(x):
  bits = plsc.bitcast(x, jnp.int32)
  e = ((bits >> 23) & 0xFF) - 127
  m = plsc.bitcast((bits & 0x7FFFFF) | 0x3F800000, jnp.float32)
  big = m > _SQRT2
  m = jnp.where(big, m * 0.5, m)
  ef = (e + jnp.where(big, 1, 0)).astype(jnp.float32)
  s = (m - 1.0) / (m + 1.0)
  s2 = s * s
  inner = 1.0 + s2 * (1.0 / 3.0 + s2 * (1.0 / 5.0 + s2 * (1.0 / 7.0)))
  return ef * _LN2 + 2.0 * s * inner


def _smooth_l1(d):
  d = jnp.abs(d)
  return jnp.where(d <= 1.0 / 9.0, 0.5 * 9.0 * d * d, d - 0.5 / 9.0)


def _sc_assign_body(anch_hbm, reg_hbm, gts_hbm, gtf_hbm,
                    code_hbm, part_hbm,
                    anch_v, reg_v, gts_v, gtf_v, code_v, acc_v):
  cid = lax.axis_index("c")
  sid = lax.axis_index("s")
  wid = sid * 2 + cid
  base = wid * _CHUNK

  pltpu.sync_copy(anch_hbm.at[:, pl.ds(base, _CHUNK)], anch_v)
  for b in range(_B):
    pltpu.sync_copy(reg_hbm.at[b].at[:, pl.ds(base, _CHUNK)], reg_v.at[b])
  pltpu.sync_copy(gts_hbm, gts_v)
  pltpu.sync_copy(gtf_hbm, gtf_v)

  zero16 = jnp.zeros((16,), jnp.float32)
  for j in range(2):
    for b in range(_B):
      acc_v[j, b] = zero16

  def body(i, carry):
    o = i * 16
    a_x0 = anch_v[0, pl.ds(o, 16)]
    a_y0 = anch_v[1, pl.ds(o, 16)]
    a_x1 = anch_v[2, pl.ds(o, 16)]
    a_y1 = anch_v[3, pl.ds(o, 16)]
    aw = jnp.abs(a_x0 - a_x1)
    ah = jnp.abs(a_y0 - a_y1)
    actr_x = a_x0 + 0.5 * aw
    actr_y = a_y0 + 0.5 * ah
    a_area = (a_x1 - a_x0) * (a_y1 - a_y0)

    best = []
    bidx = []
    for b in range(_B):
      bb = jnp.full((16,), -1.0, jnp.float32)
      bi = jnp.full((16,), b * _G, jnp.int32)
      for g in range(_G):
        k = b * _G + g
        gx0 = gts_v[k, 0]
        gy0 = gts_v[k, 1]
        gx1 = gts_v[k, 2]
        gy1 = gts_v[k, 3]
        gar = gts_v[k, 4]
        iw = jnp.maximum(jnp.minimum(a_x1, gx1) - jnp.maximum(a_x0, gx0), 0.0)
        ih = jnp.maximum(jnp.minimum(a_y1, gy1) - jnp.maximum(a_y0, gy0), 0.0)
        inter = iw * ih
        iou = inter / (a_area + gar - inter)
        m = iou > bb
        bb = jnp.where(m, iou, bb)
        bi = jnp.where(m, jnp.full((16,), k, jnp.int32), bi)
      best.append(bb)
      bidx.append(bi)

    inv_aw = 1.0 / aw
    inv_ah = 1.0 / ah
    for b in range(_B):
      pos = best[b] >= 0.5
      idx = bidx[b]
      lab = plsc.load_gather(gtf_v.at[5], [idx])
      code = jnp.where(pos, lab, jnp.where(best[b] < 0.4, -2.0, -1.0))
      code_v[b, pl.ds(o, 16)] = code
      acc_v[1, b] = acc_v[1, b] + jnp.where(pos, 1.0, 0.0)

      gx0 = plsc.load_gather(gtf_v.at[0], [idx])
      gy0 = plsc.load_gather(gtf_v.at[1], [idx])
      gx1 = plsc.load_gather(gtf_v.at[2], [idx])
      gy1 = plsc.load_gather(gtf_v.at[3], [idx])
      gw0 = gx1 - gx0
      gh0 = gy1 - gy0
      gcx = gx0 + 0.5 * gw0
      gcy = gy0 + 0.5 * gh0
      gw = jnp.maximum(gw0, 1.0)
      gh = jnp.maximum(gh0, 1.0)
      tdx = (gcx - actr_x) * inv_aw
      tdy = (gcy - actr_y) * inv_ah
      tdw = _sc_log(gw * inv_aw)
      tdh = _sc_log(gh * inv_ah)
      r0 = reg_v[b, 0, pl.ds(o, 16)]
      r1 = reg_v[b, 1, pl.ds(o, 16)]
      r2 = reg_v[b, 2, pl.ds(o, 16)]
      r3 = reg_v[b, 3, pl.ds(o, 16)]
      rl = (_smooth_l1(tdx - r0) + _smooth_l1(tdy - r1)
            + _smooth_l1(tdh - r2) + _smooth_l1(tdw - r3))
      acc_v[0, b] = acc_v[0, b] + jnp.where(pos, rl, 0.0)
    return carry

  lax.fori_loop(0, _NVEC, body, 0)

  for b in range(_B):
    pltpu.sync_copy(code_v.at[b], code_hbm.at[b, pl.ds(base, _CHUNK)])
  pltpu.sync_copy(acc_v, part_hbm.at[wid])


def _sc_assign(anch_t, reg_t, gts, gtf):
  mesh = plsc.VectorSubcoreMesh(core_axis_name="c", subcore_axis_name="s",
                                num_cores=2, num_subcores=16)
  fn = pl.kernel(
      _sc_assign_body,
      out_type=(
          jax.ShapeDtypeStruct((_B, _APAD), jnp.float32),
          jax.ShapeDtypeStruct((_NTILES, 2, _B, 16), jnp.float32),
      ),
      mesh=mesh,
      scratch_types=[
          pltpu.VMEM((4, _CHUNK), jnp.float32),
          pltpu.VMEM((_B, 4, _CHUNK), jnp.float32),
          pltpu.VMEM((_NGT, 8, 16), jnp.float32),
          pltpu.VMEM((8, _NGT), jnp.float32),
          pltpu.VMEM((_B, _CHUNK), jnp.float32),
          pltpu.VMEM((2, _B, 16), jnp.float32),
      ],
  )
  return fn(anch_t, reg_t, gts, gtf)


def _focal_body(cls_ref, code_ref, out_ref):
  j = pl.program_id(1)
  x = cls_ref[0]
  code = code_ref[0]
  c = jnp.clip(x, 1e-4, 1.0 - 1e-4)
  lane = lax.broadcasted_iota(jnp.float32, (_ABLK, _C), 1)
  t1 = (code >= 0.0) & (lane == code)
  p = jnp.where(t1, c, 1.0 - c)
  af = jnp.where(t1, 0.25, 0.75)
  val = af * jnp.square(1.0 - p) * (-jnp.log(p))
  val = jnp.where(code == -1.0, 0.0, val)
  part = jnp.sum(val)

  @pl.when(j == 0)
  def _init():
    out_ref[...] = jnp.zeros_like(out_ref)

  out_ref[...] = out_ref[...] + part


def _focal_sums(classification, code):
  return pl.pallas_call(
      _focal_body,
      grid=(_B, _NBLK),
      in_specs=[
          pl.BlockSpec((1, _ABLK, _C), lambda b, j: (b, j, 0)),
          pl.BlockSpec((1, _ABLK, 1), lambda b, j: (b, j, 0)),
      ],
      out_specs=pl.BlockSpec((1, 8, 128), lambda b, j: (b, 0, 0)),
      out_shape=jax.ShapeDtypeStruct((_B, 8, 128), jnp.float32),
      compiler_params=pltpu.CompilerParams(
          dimension_semantics=("arbitrary", "arbitrary")),
  )(classification, code)


@jax.jit
def kernel(regression, classification, anchors, gt_BB):
  f32 = jnp.float32
  a = anchors[0].astype(f32)
  anch_t = jnp.zeros((4, _APAD), f32).at[:, :_A].set(a.T)
  reg_t = jnp.zeros((_B, 4, _APAD), f32).at[:, :, :_A].set(
      jnp.transpose(regression.astype(f32), (0, 2, 1)))

  g = gt_BB.astype(f32).reshape(_NGT, 5)
  garea = (g[:, 2] - g[:, 0]) * (g[:, 3] - g[:, 1])
  gtf = jnp.concatenate(
      [g[:, 0:4].T, garea[None, :], g[:, 4][None, :],
       jnp.zeros((2, _NGT), f32)], axis=0)
  gts = jnp.broadcast_to(gtf.T[:, :, None], (_NGT, 8, 16))

  code_pad, part = _sc_assign(anch_t, reg_t, gts, gtf)

  code = code_pad[:, :_A].reshape(_B, _A, 1)
  cls_acc = _focal_sums(classification.astype(f32), code)
  cls_sums = cls_acc[:, 0, 0]

  npos = part[:, 1].sum(axis=(0, 2))
  regs = part[:, 0].sum(axis=(0, 2))
  np1 = jnp.maximum(npos, 1.0)
  cls_out = jnp.mean(cls_sums / np1, keepdims=True)
  reg_out = jnp.mean(jnp.where(npos > 0, regs / (np1 * 4.0), 0.0),
                     keepdims=True) * 50.0
  return cls_out, reg_out


# speedup vs baseline: 4.0154x; 4.0154x over previous
import functools

import jax
import jax.numpy as jnp
from jax import lax
from jax.experimental import pallas as pl
from jax.experimental.pallas import tpu as pltpu
from jax.experimental.pallas import tpu_sc as plsc

_B, _A, _C, _G = 8, 49104, 80, 20
_APAD = 49152
_NTILES = 32
_CHUNK = _APAD // _NTILES
_NVEC = _CHUNK // 16
_NGT = _B * _G
_ABLK = 1488
_NBLK = _A // _ABLK
_LN2 = 0.6931471805599453
_SQRT2 = 1.4142135623730951


def _sc_log(x):
  bits = plsc.bitcast(x, jnp.int32)
  e = ((bits >> 23) & 0xFF) - 127
  m = plsc.bitcast((bits & 0x7FFFFF) | 0x3F800000, jnp.float32)
  big = m > _SQRT2
  m = jnp.where(big, m * 0.5, m)
  ef = (e + jnp.where(big, 1, 0)).astype(jnp.float32)
  s = (m - 1.0) / (m + 1.0)
  s2 = s * s
  inner = 1.0 + s2 * (1.0 / 3.0 + s2 * (1.0 / 5.0 + s2 * (1.0 / 7.0)))
  return ef * _LN2 + 2.0 * s * inner


def _smooth_l1(d):
  d = jnp.abs(d)
  return jnp.where(d <= 1.0 / 9.0, 0.5 * 9.0 * d * d, d - 0.5 / 9.0)


def _sc_assign_body(anch_hbm, reg_hbm, gts_hbm, gtf_hbm,
                    code_hbm, part_hbm,
                    anch_v, reg_v, gts_v, gtf_v, code_v, acc_v):
  cid = lax.axis_index("c")
  sid = lax.axis_index("s")
  wid = sid * 2 + cid
  base = wid * _CHUNK

  pltpu.sync_copy(anch_hbm.at[:, pl.ds(base, _CHUNK)], anch_v)
  for b in range(_B):
    pltpu.sync_copy(reg_hbm.at[b].at[:, pl.ds(base, _CHUNK)], reg_v.at[b])
  pltpu.sync_copy(gts_hbm, gts_v)
  pltpu.sync_copy(gtf_hbm, gtf_v)

  zero16 = jnp.zeros((16,), jnp.float32)
  for j in range(2):
    for b in range(_B):
      acc_v[j, b] = zero16

  def body(i, carry):
    o = i * 16
    a_x0 = anch_v[0, pl.ds(o, 16)]
    a_y0 = anch_v[1, pl.ds(o, 16)]
    a_x1 = anch_v[2, pl.ds(o, 16)]
    a_y1 = anch_v[3, pl.ds(o, 16)]
    aw = jnp.abs(a_x0 - a_x1)
    ah = jnp.abs(a_y0 - a_y1)
    actr_x = a_x0 + 0.5 * aw
    actr_y = a_y0 + 0.5 * ah
    a_area = (a_x1 - a_x0) * (a_y1 - a_y0)

    best = []
    bidx = []
    for b in range(_B):
      bb = jnp.full((16,), -1.0, jnp.float32)
      bi = jnp.full((16,), b * _G, jnp.int32)
      for g in range(_G):
        k = b * _G + g
        gx0 = gts_v[k, 0]
        gy0 = gts_v[k, 1]
        gx1 = gts_v[k, 2]
        gy1 = gts_v[k, 3]
        gar = gts_v[k, 4]
        iw = jnp.maximum(jnp.minimum(a_x1, gx1) - jnp.maximum(a_x0, gx0), 0.0)
        ih = jnp.maximum(jnp.minimum(a_y1, gy1) - jnp.maximum(a_y0, gy0), 0.0)
        inter = iw * ih
        iou = inter / (a_area + gar - inter)
        m = iou > bb
        bb = jnp.where(m, iou, bb)
        bi = jnp.where(m, jnp.full((16,), k, jnp.int32), bi)
      best.append(bb)
      bidx.append(bi)

    inv_aw = 1.0 / aw
    inv_ah = 1.0 / ah
    for b in range(_B):
      pos = best[b] >= 0.5
      idx = bidx[b]
      def _row_gather(j, ii):
        return plsc.load_gather(
            gtf_v, [jnp.full((16,), j, jnp.int32), ii])
      lab = _row_gather(5, idx)
      code = jnp.where(pos, lab, jnp.where(best[b] < 0.4, -2.0, -1.0))
      code_v[b, pl.ds(o, 16)] = code
      acc_v[1, b] = acc_v[1, b] + jnp.where(pos, 1.0, 0.0)

      gx0 = _row_gather(0, idx)
      gy0 = _row_gather(1, idx)
      gx1 = _row_gather(2, idx)
      gy1 = _row_gather(3, idx)
      gw0 = gx1 - gx0
      gh0 = gy1 - gy0
      gcx = gx0 + 0.5 * gw0
      gcy = gy0 + 0.5 * gh0
      gw = jnp.maximum(gw0, 1.0)
      gh = jnp.maximum(gh0, 1.0)
      tdx = (gcx - actr_x) * inv_aw
      tdy = (gcy - actr_y) * inv_ah
      tdw = _sc_log(gw * inv_aw)
      tdh = _sc_log(gh * inv_ah)
      r0 = reg_v[b, 0, pl.ds(o, 16)]
      r1 = reg_v[b, 1, pl.ds(o, 16)]
      r2 = reg_v[b, 2, pl.ds(o, 16)]
      r3 = reg_v[b, 3, pl.ds(o, 16)]
      rl = (_smooth_l1(tdx - r0) + _smooth_l1(tdy - r1)
            + _smooth_l1(tdh - r2) + _smooth_l1(tdw - r3))
      acc_v[0, b] = acc_v[0, b] + jnp.where(pos, rl, 0.0)
    return carry

  lax.fori_loop(0, _NVEC, body, 0)

  for b in range(_B):
    pltpu.sync_copy(code_v.at[b], code_hbm.at[b, pl.ds(base, _CHUNK)])
  pltpu.sync_copy(acc_v, part_hbm.at[wid])


def _sc_assign(anch_t, reg_t, gts, gtf):
  mesh = plsc.VectorSubcoreMesh(core_axis_name="c", subcore_axis_name="s",
                                num_cores=2, num_subcores=16)
  fn = pl.kernel(
      _sc_assign_body,
      out_type=(
          jax.ShapeDtypeStruct((_B, _APAD), jnp.float32),
          jax.ShapeDtypeStruct((_NTILES, 2, _B, 16), jnp.float32),
      ),
      mesh=mesh,
      scratch_types=[
          pltpu.VMEM((4, _CHUNK), jnp.float32),
          pltpu.VMEM((_B, 4, _CHUNK), jnp.float32),
          pltpu.VMEM((_NGT, 8, 16), jnp.float32),
          pltpu.VMEM((8, _NGT), jnp.float32),
          pltpu.VMEM((_B, _CHUNK), jnp.float32),
          pltpu.VMEM((2, _B, 16), jnp.float32),
      ],
      compiler_params=pltpu.CompilerParams(use_tc_tiling_on_sc=False,
                                           needs_layout_passes=False),
  )
  return fn(anch_t, reg_t, gts, gtf)


def _focal_body(cls_ref, code_ref, out_ref):
  j = pl.program_id(1)
  x = cls_ref[0]
  code = code_ref[0]
  c = jnp.clip(x, 1e-4, 1.0 - 1e-4)
  lane = lax.broadcasted_iota(jnp.int32, (_ABLK, _C), 1)
  t1 = (code >= 0.0) & (lane == code.astype(jnp.int32))
  p = jnp.where(t1, c, 1.0 - c)
  af = jnp.where(t1, 0.25, 0.75)
  val = af * jnp.square(1.0 - p) * (-jnp.log(p))
  val = jnp.where(code == -1.0, 0.0, val)
  part = jnp.sum(val)

  @pl.when(j == 0)
  def _init():
    out_ref[...] = jnp.zeros_like(out_ref)

  out_ref[...] = out_ref[...] + part


def _focal_sums(classification, code):
  return pl.pallas_call(
      _focal_body,
      grid=(_B, _NBLK),
      in_specs=[
          pl.BlockSpec((1, _ABLK, _C), lambda b, j: (b, j, 0)),
          pl.BlockSpec((1, _ABLK, 1), lambda b, j: (b, j, 0)),
      ],
      out_specs=pl.BlockSpec((1, 8, 128), lambda b, j: (b, 0, 0)),
      out_shape=jax.ShapeDtypeStruct((_B, 8, 128), jnp.float32),
      compiler_params=pltpu.CompilerParams(
          dimension_semantics=("arbitrary", "arbitrary")),
  )(classification, code)


@jax.jit
def kernel(regression, classification, anchors, gt_BB):
  f32 = jnp.float32
  a = anchors[0].astype(f32)
  anch_t = jnp.zeros((4, _APAD), f32).at[:, :_A].set(a.T)
  reg_t = jnp.zeros((_B, 4, _APAD), f32).at[:, :, :_A].set(
      jnp.transpose(regression.astype(f32), (0, 2, 1)))

  g = gt_BB.astype(f32).reshape(_NGT, 5)
  garea = (g[:, 2] - g[:, 0]) * (g[:, 3] - g[:, 1])
  gtf = jnp.concatenate(
      [g[:, 0:4].T, garea[None, :], g[:, 4][None, :],
       jnp.zeros((2, _NGT), f32)], axis=0)
  gts = jnp.broadcast_to(gtf.T[:, :, None], (_NGT, 8, 16))

  code_pad, part = _sc_assign(anch_t, reg_t, gts, gtf)

  code = code_pad[:, :_A].reshape(_B, _A, 1)
  cls_acc = _focal_sums(classification.astype(f32), code)
  cls_sums = cls_acc[:, 0, 0]

  npos = part[:, 1].sum(axis=(0, 2))
  regs = part[:, 0].sum(axis=(0, 2))
  np1 = jnp.maximum(npos, 1.0)
  cls_out = jnp.mean(cls_sums / np1, keepdims=True)
  reg_out = jnp.mean(jnp.where(npos > 0, regs / (np1 * 4.0), 0.0),
                     keepdims=True) * 50.0
  return cls_out, reg_out
